# Initial kernel scaffold; baseline (speedup 1.0000x reference)
import jax
import jax.numpy as jnp
from jax.experimental import pallas as pl


def kernel(x, edge_index, edge_attr, Wl0, bl0, Wr0, br0, att0, We0, cb0, lg0, lb0, Wl1, bl1, Wr1, br1, att1, We1, cb1, lg1, lb1, Wl2, bl2, Wr2, br2, att2, We2, cb2, lg2, lb2, Wp, bp):
    raise NotImplementedError("write your pallas kernel here")


# speedup vs baseline: 13.2701x; 13.2701x over previous
import functools

import jax
import jax.numpy as jnp
from jax import lax
from jax.experimental import pallas as pl
from jax.experimental.pallas import tpu as pltpu
from jax.experimental.pallas import tpu_sc as plsc

N = 10000
E = 320000
D = 128
H = 8
C = 16
ED = 4

NC = 2
NS = 16
NW = NC * NS

NP = 10240
CHUNK = 64
ET = E + N
EPAD = 331776
ECH = EPAD // CHUNK
CPW = ECH // NW
DROWS = NP // 16
NPD = NP + DROWS

BLK = 512
GRID = NP // BLK


def _mm_fill_body(x_ref, ea_ref, wl_ref, bl_ref, wr_ref, br_ref,
                  xl_ref, xr_ref, fill_ref):
    i = pl.program_id(0)
    xb = x_ref[...]
    xl_ref[...] = jnp.dot(xb, wl_ref[...], preferred_element_type=jnp.float32) + bl_ref[...]
    xr_ref[...] = jnp.dot(xb, wr_ref[...], preferred_element_type=jnp.float32) + br_ref[...]

    @pl.when(i == 0)
    def _():
        fill_ref[...] = jnp.zeros_like(fill_ref)

    colsum = jnp.sum(ea_ref[...], axis=0, keepdims=True)
    fill_ref[...] += jnp.broadcast_to(colsum, fill_ref.shape)

    @pl.when(i == GRID - 1)
    def _():
        s = fill_ref[0:1, :]
        g = (lax.broadcasted_iota(jnp.int32, (128, ED), 0) % ED
             == lax.broadcasted_iota(jnp.int32, (128, ED), 1)).astype(jnp.float32)
        f = jnp.dot(s, g, preferred_element_type=jnp.float32) / float(E)
        f8 = jnp.broadcast_to(f, (8, ED))
        fill_ref[...] = jnp.concatenate(
            [f8, jnp.zeros((8, 128 - ED), jnp.float32)], axis=1)


def _matmuls_and_fill(x_pad, ea_r, wl, bl, wr, br):
    return pl.pallas_call(
        _mm_fill_body,
        grid=(GRID,),
        in_specs=[
            pl.BlockSpec((BLK, D), lambda i: (i, 0)),
            pl.BlockSpec((BLK, 128), lambda i: (i, 0)),
            pl.BlockSpec((D, D), lambda i: (0, 0)),
            pl.BlockSpec((1, D), lambda i: (0, 0)),
            pl.BlockSpec((D, D), lambda i: (0, 0)),
            pl.BlockSpec((1, D), lambda i: (0, 0)),
        ],
        out_specs=[
            pl.BlockSpec((BLK, D), lambda i: (i, 0)),
            pl.BlockSpec((BLK, D), lambda i: (i, 0)),
            pl.BlockSpec((8, 128), lambda i: (0, 0)),
        ],
        out_shape=[
            jax.ShapeDtypeStruct((NP, D), jnp.float32),
            jax.ShapeDtypeStruct((NP, D), jnp.float32),
            jax.ShapeDtypeStruct((8, 128), jnp.float32),
        ],
    )(x_pad, ea_r, wl, bl, wr, br)


def _head_expand():
    return (lax.broadcasted_iota(jnp.int32, (H, H * C), 1) // C
            == lax.broadcasted_iota(jnp.int32, (H, H * C), 0)).astype(jnp.float32)


def _normalize(num_ref, den_ref):
    num = num_ref[0] + num_ref[1]
    den = jnp.sum(den_ref[...], axis=0)
    den128 = jnp.dot(den, _head_expand(), preferred_element_type=jnp.float32)
    return num / (den128 + 1e-16)


def _ln(h, g, b):
    mu = jnp.mean(h, axis=-1, keepdims=True)
    var = jnp.mean((h - mu) * (h - mu), axis=-1, keepdims=True)
    return (h - mu) * lax.rsqrt(var + 1e-5) * g + b


def _elu(x):
    return jnp.where(x > 0, x, jnp.exp(jnp.minimum(x, 0.0)) - 1.0)


def _combine_mm_body(num_ref, den_ref, hprev_ref, cb_ref, lg_ref, lb_ref,
                     wl_ref, bl_ref, wr_ref, br_ref,
                     h_ref, xl_ref, xr_ref):
    hh = _normalize(num_ref, den_ref) + cb_ref[...]
    hh = _elu(hh)
    hh = _ln(hh, lg_ref[...], lb_ref[...])
    hh = hh + hprev_ref[...]
    h_ref[...] = hh
    xl_ref[...] = jnp.dot(hh, wl_ref[...], preferred_element_type=jnp.float32) + bl_ref[...]
    xr_ref[...] = jnp.dot(hh, wr_ref[...], preferred_element_type=jnp.float32) + br_ref[...]


def _combine_and_matmuls(num, den, hprev, cb, lg, lb, wl, bl, wr, br):
    return pl.pallas_call(
        _combine_mm_body,
        grid=(GRID,),
        in_specs=[
            pl.BlockSpec((2, BLK, D), lambda i: (0, i, 0)),
            pl.BlockSpec((NC, BLK, H), lambda i: (0, i, 0)),
            pl.BlockSpec((BLK, D), lambda i: (i, 0)),
            pl.BlockSpec((1, D), lambda i: (0, 0)),
            pl.BlockSpec((1, D), lambda i: (0, 0)),
            pl.BlockSpec((1, D), lambda i: (0, 0)),
            pl.BlockSpec((D, D), lambda i: (0, 0)),
            pl.BlockSpec((1, D), lambda i: (0, 0)),
            pl.BlockSpec((D, D), lambda i: (0, 0)),
            pl.BlockSpec((1, D), lambda i: (0, 0)),
        ],
        out_specs=[
            pl.BlockSpec((BLK, D), lambda i: (i, 0)),
            pl.BlockSpec((BLK, D), lambda i: (i, 0)),
            pl.BlockSpec((BLK, D), lambda i: (i, 0)),
        ],
        out_shape=[
            jax.ShapeDtypeStruct((NP, D), jnp.float32),
            jax.ShapeDtypeStruct((NP, D), jnp.float32),
            jax.ShapeDtypeStruct((NP, D), jnp.float32),
        ],
    )(num, den, hprev, cb, lg, lb, wl, bl, wr, br)


def _final_body(num_ref, den_ref, cb_ref, lg_ref, lb_ref, wp_ref, bp_ref,
                out_ref):
    hh = _normalize(num_ref, den_ref)
    m = (lax.broadcasted_iota(jnp.int32, (H * C, C), 0) % C
         == lax.broadcasted_iota(jnp.int32, (H * C, C), 1)).astype(jnp.float32) / float(H)
    h2 = jnp.dot(hh, m, preferred_element_type=jnp.float32) + cb_ref[...]
    h2 = _elu(h2)
    h2 = _ln(h2, lg_ref[...], lb_ref[...])
    out_ref[...] = jnp.dot(h2, wp_ref[...], preferred_element_type=jnp.float32) + bp_ref[...]


def _final(num, den, cb, lg, lb, wp, bp):
    return pl.pallas_call(
        _final_body,
        grid=(GRID,),
        in_specs=[
            pl.BlockSpec((2, BLK, D), lambda i: (0, i, 0)),
            pl.BlockSpec((NC, BLK, H), lambda i: (0, i, 0)),
            pl.BlockSpec((1, C), lambda i: (0, 0)),
            pl.BlockSpec((1, C), lambda i: (0, 0)),
            pl.BlockSpec((1, C), lambda i: (0, 0)),
            pl.BlockSpec((C, D), lambda i: (0, 0)),
            pl.BlockSpec((1, D), lambda i: (0, 0)),
        ],
        out_specs=pl.BlockSpec((BLK, D), lambda i: (i, 0)),
        out_shape=jax.ShapeDtypeStruct((NP, D), jnp.float32),
    )(num, den, cb, lg, lb, wp, bp)


def _sc_edge_body(xl_hbm, xr_hbm, src_hbm, dst_hbm, ea_hbm, we_hbm, att_hbm,
                  num_hbm, den_hbm,
                  srcbuf, dstbuf, idx2buf, eabuf, xsbuf, xdbuf, valbuf,
                  val2buf, webuf, attbuf, acc, sem1, sem2):
    c = lax.axis_index("c")
    s = lax.axis_index("s")
    wid = c * NS + s

    pltpu.sync_copy(we_hbm, webuf)
    pltpu.sync_copy(att_hbm, attbuf)

    zeros16 = jnp.zeros((16,), jnp.float32)

    def _zrow(e, _):
        for j in range(D // 16):
            valbuf[e, pl.ds(16 * j, 16)] = zeros16
        return 0

    lax.fori_loop(0, CHUNK, _zrow, 0)
    rows_per_tile = NPD // NS
    tbase = s * rows_per_tile
    for r in range(rows_per_tile // CHUNK):
        pltpu.sync_copy(valbuf, acc.at[pl.ds(tbase + r * CHUNK, CHUNK)])
    rem = rows_per_tile % CHUNK
    if rem:
        pltpu.sync_copy(valbuf.at[pl.ds(0, rem)],
                        acc.at[pl.ds(tbase + rows_per_tile - rem, rem)])
    plsc.subcore_barrier()

    lane = lax.iota(jnp.int32, 16)
    dnums = lax.GatherDimensionNumbers(
        offset_dims=(), collapsed_slice_dims=(0,), start_index_map=(0,))

    def _lanetake(v, idx):
        return lax.gather(v, idx[:, None], dnums, (1,),
                          mode=lax.GatherScatterMode.PROMISE_IN_BOUNDS)

    def _lanesum(v):
        for sh in (8, 4, 2, 1):
            v = v + _lanetake(v, lane ^ sh)
        return v

    def _edge(e, _):
        ea4 = eabuf[pl.ds(ED * e, 16)]
        a0 = ea4[0]
        a1 = ea4[1]
        a2 = ea4[2]
        a3 = ea4[3]
        dvec = jnp.zeros((16,), jnp.float32)
        for h in range(H):
            em = (a0 * webuf[0, pl.ds(16 * h, 16)]
                  + a1 * webuf[1, pl.ds(16 * h, 16)]
                  + a2 * webuf[2, pl.ds(16 * h, 16)]
                  + a3 * webuf[3, pl.ds(16 * h, 16)])
            xs = xsbuf[e, pl.ds(16 * h, 16)]
            z = xs + xdbuf[e, pl.ds(16 * h, 16)] + em
            z = jnp.maximum(z, 0.2 * z)
            alpha = _lanesum(z * attbuf[h, :])
            ex = jnp.exp(alpha)
            valbuf[e, pl.ds(16 * h, 16)] = xs * ex
            dvec = jnp.where(lane == h, ex, dvec)
        for j in range(D // 16):
            val2buf[e, pl.ds(16 * j, 16)] = zeros16
        base = (e >> 4) << 4
        grp = dstbuf[pl.ds(base, 16)]
        dsc = _lanetake(grp, jnp.full((16,), e - base, jnp.int32))[0]
        off = (dsc & 15) << 3
        st = off & 15
        hv = lane - st
        val2 = jnp.where((hv >= 0) & (hv < H), _lanetake(dvec, hv & 15), 0.0)
        val2buf[e, pl.ds((off >> 4) << 4, 16)] = val2
        return 0

    def _chunk(t, _):
        j = wid * CPW + t
        pltpu.sync_copy(src_hbm.at[j], srcbuf)
        pltpu.sync_copy(dst_hbm.at[j], dstbuf)
        pltpu.sync_copy(ea_hbm.at[j], eabuf.at[pl.ds(0, CHUNK * ED)])
        cp1 = pltpu.async_copy(xl_hbm.at[srcbuf], xsbuf, sem1)
        cp2 = pltpu.async_copy(xr_hbm.at[dstbuf], xdbuf, sem2)
        cp1.wait()
        cp2.wait()
        for g in range(CHUNK // 16):
            dv = dstbuf[pl.ds(16 * g, 16)]
            idx2buf[pl.ds(16 * g, 16)] = NP + (dv >> 4)
        lax.fori_loop(0, CHUNK, _edge, 0)
        pltpu.sync_copy(valbuf, acc.at[dstbuf], add=True)
        pltpu.sync_copy(val2buf, acc.at[idx2buf], add=True)
        return 0

    lax.fori_loop(0, CPW, _chunk, 0)

    plsc.subcore_barrier()
    pltpu.sync_copy(acc.at[pl.ds(s * (NP // NS), NP // NS)],
                    num_hbm.at[c, pl.ds(s * (NP // NS), NP // NS)])
    pltpu.sync_copy(acc.at[pl.ds(NP + s * (DROWS // NS), DROWS // NS)],
                    den_hbm.at[c, pl.ds(s * (DROWS // NS), DROWS // NS)])


@functools.cache
def _make_sc_edge_pass():
    @functools.partial(
        pl.kernel,
        out_type=[
            jax.ShapeDtypeStruct((NC, NP, D), jnp.float32),
            jax.ShapeDtypeStruct((NC, DROWS, D), jnp.float32),
        ],
        mesh=plsc.VectorSubcoreMesh(core_axis_name="c", subcore_axis_name="s"),
        compiler_params=pltpu.CompilerParams(needs_layout_passes=False),
        scratch_types=[
            pltpu.VMEM((CHUNK,), jnp.int32),
            pltpu.VMEM((CHUNK,), jnp.int32),
            pltpu.VMEM((CHUNK,), jnp.int32),
            pltpu.VMEM((CHUNK * ED + 16,), jnp.float32),
            pltpu.VMEM((CHUNK, D), jnp.float32),
            pltpu.VMEM((CHUNK, D), jnp.float32),
            pltpu.VMEM((CHUNK, D), jnp.float32),
            pltpu.VMEM((CHUNK, D), jnp.float32),
            pltpu.VMEM((ED, D), jnp.float32),
            pltpu.VMEM((H, C), jnp.float32),
            pltpu.VMEM_SHARED((NPD, D), jnp.float32),
            pltpu.SemaphoreType.DMA,
            pltpu.SemaphoreType.DMA,
        ],
    )
    def _sc_edge_pass(xl_hbm, xr_hbm, src_hbm, dst_hbm, ea_hbm, we_hbm,
                      att_hbm, num_hbm, den_hbm,
                      srcbuf, dstbuf, idx2buf, eabuf, xsbuf, xdbuf, valbuf,
                      val2buf, webuf, attbuf, acc, sem1, sem2):
        _sc_edge_body(xl_hbm, xr_hbm, src_hbm, dst_hbm, ea_hbm, we_hbm,
                      att_hbm, num_hbm, den_hbm,
                      srcbuf, dstbuf, idx2buf, eabuf, xsbuf, xdbuf, valbuf,
                      val2buf, webuf, attbuf, acc, sem1, sem2)

    return _sc_edge_pass


def kernel(x, edge_index, edge_attr,
           Wl0, bl0, Wr0, br0, att0, We0, cb0, lg0, lb0,
           Wl1, bl1, Wr1, br1, att1, We1, cb1, lg1, lb1,
           Wl2, bl2, Wr2, br2, att2, We2, cb2, lg2, lb2,
           Wp, bp):
    f32 = jnp.float32
    x_pad = jnp.pad(x, ((0, NP - N), (0, 0)))
    ea_r = jnp.pad(edge_attr.reshape(E * ED // 128, 128),
                   ((0, NP - E * ED // 128), (0, 0)))

    loop = jnp.arange(N, dtype=edge_index.dtype)
    src_full = jnp.concatenate(
        [edge_index[0], loop,
         jnp.zeros((EPAD - ET,), edge_index.dtype)]).reshape(ECH, CHUNK)
    dst_full = jnp.concatenate(
        [edge_index[1], loop,
         jnp.full((EPAD - ET,), N, edge_index.dtype)]).reshape(ECH, CHUNK)

    xl, xr, fill8 = _matmuls_and_fill(
        x_pad, ea_r, Wl0, bl0.reshape(1, D), Wr0, br0.reshape(1, D))
    fill = fill8[0:1, 0:ED]

    ea_full = jnp.concatenate(
        [edge_attr, jnp.broadcast_to(fill, (N, ED)),
         jnp.zeros((EPAD - ET, ED), f32)]).reshape(ECH, CHUNK * ED)

    layers = [
        (We0, att0, cb0, lg0, lb0, Wl1, bl1, Wr1, br1),
        (We1, att1, cb1, lg1, lb1, Wl2, bl2, Wr2, br2),
    ]
    hprev = x_pad
    for (we, att, cb, lg, lb, wln, bln, wrn, brn) in layers:
        num, den_p = _make_sc_edge_pass()(
            xl, xr, src_full, dst_full, ea_full, we, att)
        den = den_p.reshape(NC, NP, H)
        hprev, xl, xr = _combine_and_matmuls(
            num, den, hprev, cb.reshape(1, D), lg.reshape(1, D),
            lb.reshape(1, D), wln, bln.reshape(1, D), wrn, brn.reshape(1, D))

    num, den_p = _make_sc_edge_pass()(
        xl, xr, src_full, dst_full, ea_full, We2, att2)
    den = den_p.reshape(NC, NP, H)
    out = _final(num, den, cb2.reshape(1, C), lg2.reshape(1, C),
                 lb2.reshape(1, C), Wp, bp.reshape(1, D))
    return out[:N]
